# baseline (speedup 1.0000x reference)
import functools

import jax
import jax.numpy as jnp
from jax import lax
from jax.experimental import pallas as pl
from jax.experimental.pallas import tpu as pltpu
from jax.experimental.pallas import tpu_sc as plsc

N = 10000
E = 320000
B = 256
ND = 128
ED = 16
GD = 128
HID = 4 * GD

NC, NS, L = 2, 16, 16
NW = NC * NS
CH = 128
CHE = 256
ECHUNKS = E // CHE
XCHUNKS = N // CH
XTAIL = N - XCHUNKS * CH
ACC = 3 * B * ED


def _sc_body(src_hbm, attr_hbm, batch_hbm, x_hbm, z128_hbm,
             xsum_hbm, acc_hbm,
             batch_v, src_v0, src_v1, attr_v0, attr_v1, x_v,
             bidx_v, bidx_t, x_t, acc_v,
             sem_s0, sem_s1, sem_a0, sem_a1,
             xsum_s):
    cid = lax.axis_index("c")
    sid = lax.axis_index("s")
    wid = sid * NC + cid
    lane = lax.broadcasted_iota(jnp.int32, (L,), 0)
    ones16 = jnp.full((L,), 1.0, jnp.float32)
    src_b = (src_v0, src_v1)
    attr_b = (attr_v0, attr_v1)
    sem_s = (sem_s0, sem_s1)
    sem_a = (sem_a0, sem_a1)

    pltpu.sync_copy(batch_hbm, batch_v)

    def zstep(i, c):
        acc_v[pl.ds(i * L, L)] = jnp.zeros((L,), jnp.float32)
        return c
    lax.fori_loop(0, ACC // L, zstep, 0)

    @pl.when(sid == 0)
    def _():
        pltpu.sync_copy(z128_hbm, xsum_s)

    plsc.subcore_barrier()

    ne = ECHUNKS // NW + (wid < ECHUNKS % NW).astype(jnp.int32)

    def estart(k, b):
        base = (wid + k * NW) * CHE
        pltpu.async_copy(src_hbm.at[pl.ds(base, CHE)], src_b[b], sem_s[b])
        pltpu.async_copy(attr_hbm.at[:, pl.ds(base, CHE)],
                         attr_b[b].at[:, pl.ds(0, CHE)], sem_a[b])

    estart(0, 0)
    estart(1, 1)

    def echunk(i, carry):
        for b in range(2):
            k = i * 2 + b

            @pl.when(k < ne)
            def _():
                pltpu.make_async_copy(src_hbm.at[pl.ds(0, CHE)],
                                      src_b[b], sem_s[b]).wait()
                pltpu.make_async_copy(attr_hbm.at[:, pl.ds(0, CHE)],
                                      attr_b[b].at[:, pl.ds(0, CHE)],
                                      sem_a[b]).wait()
                for g in range(CHE // L):
                    sv = src_b[b][pl.ds(g * L, L)]
                    seg = plsc.load_gather(batch_v, [sv])
                    seg16 = seg * ED
                    plsc.addupdate_scatter(acc_v, [seg16 + lane + B * ED],
                                           ones16)
                    for j in range(L):
                        e = g * L + j
                        vals = plsc.load_gather(
                            attr_b[b], [lane, jnp.full((L,), e, jnp.int32)])
                        bj = seg16.at[jnp.full((L,), j, jnp.int32)].get(
                            mode="promise_in_bounds") + lane
                        plsc.addupdate_scatter(acc_v, [bj], vals)

                @pl.when(k + 2 < ne)
                def _():
                    estart(k + 2, b)
        return carry

    lax.fori_loop(0, (ne + 1) // 2, echunk, 0)

    def xchunk(i, carry):
        base = (wid + i * NW) * CH
        pltpu.sync_copy(batch_hbm.at[pl.ds(base, CH)], bidx_v)
        pltpu.sync_copy(x_hbm.at[pl.ds(base, CH)], x_v)
        pltpu.sync_copy(x_v, xsum_s.at[bidx_v], add=True)
        for g in range(CH // L):
            bv = bidx_v[pl.ds(g * L, L)]
            plsc.addupdate_scatter(acc_v, [bv * ED + lane + 2 * B * ED], ones16)
        return carry

    nx = XCHUNKS // NW + (wid < XCHUNKS % NW).astype(jnp.int32)
    lax.fori_loop(0, nx, xchunk, 0)

    @pl.when(wid == 0)
    def _():
        pltpu.sync_copy(batch_hbm.at[pl.ds(XCHUNKS * CH, XTAIL)], bidx_t)
        pltpu.sync_copy(x_hbm.at[pl.ds(XCHUNKS * CH, XTAIL)], x_t)
        pltpu.sync_copy(x_t, xsum_s.at[bidx_t], add=True)
        bv = bidx_t[...]
        plsc.addupdate_scatter(acc_v, [bv * ED + lane + 2 * B * ED], ones16)

    pltpu.sync_copy(acc_v, acc_hbm.at[wid])

    plsc.subcore_barrier()

    @pl.when(sid == 0)
    def _():
        pltpu.sync_copy(xsum_s, xsum_hbm.at[cid])


def _mlp_body(acc_ref, xsum_ref, u_ref,
              W1_ref, b1_ref, W2_ref, b2_ref, lnw_ref, lnb_ref, out_ref):
    xsum = xsum_ref[0] + xsum_ref[1]
    esum = jnp.sum(acc_ref[:, 0], axis=0)
    ecnt = jnp.maximum(
        jnp.sum(jnp.sum(acc_ref[:, 1], axis=0), axis=1, keepdims=True), 1.0)
    xcnt = jnp.maximum(
        jnp.sum(jnp.sum(acc_ref[:, 2], axis=0), axis=1, keepdims=True), 1.0)
    e_mean = esum / ecnt
    x_mean = xsum / xcnt
    u = u_ref[...]
    h = (jnp.dot(u, W1_ref[0:GD, :], preferred_element_type=jnp.float32)
         + jnp.dot(e_mean, W1_ref[GD:GD + ED, :], preferred_element_type=jnp.float32)
         + jnp.dot(x_mean, W1_ref[GD + ED:GD + ED + ND, :], preferred_element_type=jnp.float32)
         + b1_ref[...])
    h = jnp.maximum(h, 0.0)
    o = jnp.dot(h, W2_ref[...], preferred_element_type=jnp.float32) + b2_ref[...] + u
    mu = jnp.mean(o, axis=-1, keepdims=True)
    d = o - mu
    var = jnp.mean(d * d, axis=-1, keepdims=True)
    out_ref[...] = d * jax.lax.rsqrt(var + 1e-5) * lnw_ref[...] + lnb_ref[...]


def _sc_stage(src, edge_attr_t, batch, x):
    z128 = jnp.zeros((B, ND), jnp.float32)
    mesh = plsc.VectorSubcoreMesh(core_axis_name="c", subcore_axis_name="s",
                                  num_cores=NC, num_subcores=NS)
    f32 = jnp.float32
    sc = functools.partial(
        pl.kernel,
        out_type=[jax.ShapeDtypeStruct((NC, B, ND), f32),
                  jax.ShapeDtypeStruct((NW, ACC), f32)],
        mesh=mesh,
        compiler_params=pltpu.CompilerParams(needs_layout_passes=False),
        scratch_types=[
            pltpu.VMEM((N,), jnp.int32),
            pltpu.VMEM((CHE,), jnp.int32),
            pltpu.VMEM((CHE,), jnp.int32),
            pltpu.VMEM((ED, CHE + 1), f32),
            pltpu.VMEM((ED, CHE + 1), f32),
            pltpu.VMEM((CH, ND), f32),
            pltpu.VMEM((CH,), jnp.int32),
            pltpu.VMEM((XTAIL,), jnp.int32),
            pltpu.VMEM((XTAIL, ND), f32),
            pltpu.VMEM((ACC,), f32),
            pltpu.SemaphoreType.DMA,
            pltpu.SemaphoreType.DMA,
            pltpu.SemaphoreType.DMA,
            pltpu.SemaphoreType.DMA,
            pltpu.VMEM_SHARED((B, ND), f32),
        ],
    )(_sc_body)
    return sc(src, edge_attr_t, batch, x, z128)


def kernel(x, edge_index, edge_attr, u, batch, W1, b1, W2, b2, ln_w, ln_b):
    xsum, acc = _sc_stage(edge_index[0], edge_attr.T, batch, x)

    out = pl.pallas_call(
        _mlp_body,
        out_shape=jax.ShapeDtypeStruct((B, GD), jnp.float32),
    )(acc.reshape(NW, 3, B, ED), xsum, u,
      W1, b1.reshape(1, HID), W2, b2.reshape(1, GD),
      ln_w.reshape(1, GD), ln_b.reshape(1, GD))
    return out


# speedup vs baseline: 1.2297x; 1.2297x over previous
import functools

import jax
import jax.numpy as jnp
from jax import lax
from jax.experimental import pallas as pl
from jax.experimental.pallas import tpu as pltpu
from jax.experimental.pallas import tpu_sc as plsc

N = 10000
E = 320000
B = 256
ND = 128
ED = 16
GD = 128
HID = 4 * GD

NC, NS, L = 2, 16, 16
NW = NC * NS
CH = 128
CHE = 256
ECHUNKS = E // CHE
XCHUNKS = N // CH
XTAIL = N - XCHUNKS * CH
ACC = 3 * B * ED


def _sc_body(src_hbm, attr_hbm, batch_hbm, x_hbm, z128_hbm,
             xsum_hbm, acc_hbm,
             batch_v, src_v0, src_v1, attr_v0, attr_v1, x_v,
             bidx_v, bidx_t, x_t, acc_v,
             sem_s0, sem_s1, sem_a0, sem_a1,
             xsum_s):
    cid = lax.axis_index("c")
    sid = lax.axis_index("s")
    wid = sid * NC + cid
    lane = lax.broadcasted_iota(jnp.int32, (L,), 0)
    ones16 = jnp.full((L,), 1.0, jnp.float32)
    src_b = (src_v0, src_v1)
    attr_b = (attr_v0, attr_v1)
    sem_s = (sem_s0, sem_s1)
    sem_a = (sem_a0, sem_a1)

    pltpu.sync_copy(batch_hbm, batch_v)

    def zstep(i, c):
        acc_v[pl.ds(i * L, L)] = jnp.zeros((L,), jnp.float32)
        return c
    lax.fori_loop(0, ACC // L, zstep, 0)

    @pl.when(sid == 0)
    def _():
        pltpu.sync_copy(z128_hbm, xsum_s)

    plsc.subcore_barrier()

    ne = ECHUNKS // NW + (wid < ECHUNKS % NW).astype(jnp.int32)

    def estart(k, b):
        base = (wid + k * NW) * CHE
        pltpu.async_copy(src_hbm.at[pl.ds(base, CHE)], src_b[b], sem_s[b])
        pltpu.async_copy(attr_hbm.at[pl.ds(base, CHE)], attr_b[b], sem_a[b])

    estart(0, 0)
    estart(1, 1)

    def echunk(i, carry):
        for b in range(2):
            k = i * 2 + b

            @pl.when(k < ne)
            def _():
                pltpu.make_async_copy(src_hbm.at[pl.ds(0, CHE)],
                                      src_b[b], sem_s[b]).wait()
                pltpu.make_async_copy(attr_hbm.at[pl.ds(0, CHE)],
                                      attr_b[b], sem_a[b]).wait()
                for g in range(CHE // L):
                    sv = src_b[b][pl.ds(g * L, L)]
                    seg = plsc.load_gather(batch_v, [sv])
                    seg16 = seg * ED
                    plsc.addupdate_scatter(acc_v, [seg16 + lane + B * ED],
                                           ones16)
                    for j in range(L):
                        bj = seg16.at[jnp.full((L,), j, jnp.int32)].get(
                            mode="promise_in_bounds") + lane
                        plsc.addupdate_scatter(acc_v, [bj],
                                               attr_b[b][g * L + j, :])

                @pl.when(k + 2 < ne)
                def _():
                    estart(k + 2, b)
        return carry

    lax.fori_loop(0, (ne + 1) // 2, echunk, 0)

    def xchunk(i, carry):
        base = (wid + i * NW) * CH
        pltpu.sync_copy(batch_hbm.at[pl.ds(base, CH)], bidx_v)
        pltpu.sync_copy(x_hbm.at[pl.ds(base, CH)], x_v)
        pltpu.sync_copy(x_v, xsum_s.at[bidx_v], add=True)
        for g in range(CH // L):
            bv = bidx_v[pl.ds(g * L, L)]
            plsc.addupdate_scatter(acc_v, [bv * ED + lane + 2 * B * ED], ones16)
        return carry

    nx = XCHUNKS // NW + (wid < XCHUNKS % NW).astype(jnp.int32)
    lax.fori_loop(0, nx, xchunk, 0)

    @pl.when(wid == 0)
    def _():
        pltpu.sync_copy(batch_hbm.at[pl.ds(XCHUNKS * CH, XTAIL)], bidx_t)
        pltpu.sync_copy(x_hbm.at[pl.ds(XCHUNKS * CH, XTAIL)], x_t)
        pltpu.sync_copy(x_t, xsum_s.at[bidx_t], add=True)
        bv = bidx_t[...]
        plsc.addupdate_scatter(acc_v, [bv * ED + lane + 2 * B * ED], ones16)

    pltpu.sync_copy(acc_v, acc_hbm.at[wid])

    plsc.subcore_barrier()

    @pl.when(sid == 0)
    def _():
        pltpu.sync_copy(xsum_s, xsum_hbm.at[cid])


def _reduce_body(acc_ref, out_ref):
    out_ref[...] = jnp.sum(acc_ref[...], axis=0, keepdims=True)


def _mlp_body(esum_ref, ecnt_ref, xcnt_ref, xsum_ref, u_ref,
              W1_ref, b1_ref, W2_ref, b2_ref, lnw_ref, lnb_ref, out_ref):
    xsum = xsum_ref[0] + xsum_ref[1]
    ecnt = jnp.maximum(jnp.sum(ecnt_ref[...], axis=1, keepdims=True), 1.0)
    xcnt = jnp.maximum(jnp.sum(xcnt_ref[...], axis=1, keepdims=True), 1.0)
    e_mean = esum_ref[...] / ecnt
    x_mean = xsum / xcnt
    u = u_ref[...]
    h = (jnp.dot(u, W1_ref[0:GD, :], preferred_element_type=jnp.float32)
         + jnp.dot(e_mean, W1_ref[GD:GD + ED, :], preferred_element_type=jnp.float32)
         + jnp.dot(x_mean, W1_ref[GD + ED:GD + ED + ND, :], preferred_element_type=jnp.float32)
         + b1_ref[...])
    h = jnp.maximum(h, 0.0)
    o = jnp.dot(h, W2_ref[...], preferred_element_type=jnp.float32) + b2_ref[...] + u
    mu = jnp.mean(o, axis=-1, keepdims=True)
    d = o - mu
    var = jnp.mean(d * d, axis=-1, keepdims=True)
    out_ref[...] = d * jax.lax.rsqrt(var + 1e-5) * lnw_ref[...] + lnb_ref[...]


def _sc_stage(src, edge_attr, batch, x):
    z128 = jnp.zeros((B, ND), jnp.float32)
    mesh = plsc.VectorSubcoreMesh(core_axis_name="c", subcore_axis_name="s",
                                  num_cores=NC, num_subcores=NS)
    f32 = jnp.float32
    sc = functools.partial(
        pl.kernel,
        out_type=[jax.ShapeDtypeStruct((NC, B, ND), f32),
                  jax.ShapeDtypeStruct((NW, ACC), f32)],
        mesh=mesh,
        compiler_params=pltpu.CompilerParams(needs_layout_passes=False),
        scratch_types=[
            pltpu.VMEM((N,), jnp.int32),
            pltpu.VMEM((CHE,), jnp.int32),
            pltpu.VMEM((CHE,), jnp.int32),
            pltpu.VMEM((CHE, ED), f32),
            pltpu.VMEM((CHE, ED), f32),
            pltpu.VMEM((CH, ND), f32),
            pltpu.VMEM((CH,), jnp.int32),
            pltpu.VMEM((XTAIL,), jnp.int32),
            pltpu.VMEM((XTAIL, ND), f32),
            pltpu.VMEM((ACC,), f32),
            pltpu.SemaphoreType.DMA,
            pltpu.SemaphoreType.DMA,
            pltpu.SemaphoreType.DMA,
            pltpu.SemaphoreType.DMA,
            pltpu.VMEM_SHARED((B, ND), f32),
        ],
    )(_sc_body)
    return sc(src, edge_attr, batch, x, z128)


def kernel(x, edge_index, edge_attr, u, batch, W1, b1, W2, b2, ln_w, ln_b):
    xsum, acc = _sc_stage(edge_index[0], edge_attr, batch, x)

    red = pl.pallas_call(
        _reduce_body,
        out_shape=jax.ShapeDtypeStruct((1, ACC), jnp.float32),
    )(acc)
    esum = red[0, 0:B * ED].reshape(B, ED)
    ecnt = red[0, B * ED:2 * B * ED].reshape(B, ED)
    xcnt = red[0, 2 * B * ED:3 * B * ED].reshape(B, ED)

    out = pl.pallas_call(
        _mlp_body,
        out_shape=jax.ShapeDtypeStruct((B, GD), jnp.float32),
    )(esum, ecnt, xcnt, xsum, u,
      W1, b1.reshape(1, HID), W2, b2.reshape(1, GD),
      ln_w.reshape(1, GD), ln_b.reshape(1, GD))
    return out


# speedup vs baseline: 2.0504x; 1.6675x over previous
import functools

import jax
import jax.numpy as jnp
from jax import lax
from jax.experimental import pallas as pl
from jax.experimental.pallas import tpu as pltpu
from jax.experimental.pallas import tpu_sc as plsc

N = 10000
E = 320000
B = 256
ND = 128
ED = 16
GD = 128
HID = 4 * GD

NC, NS, L = 2, 16, 16
NW = NC * NS
CH = 128
CHE = 256
ECHUNKS = E // CHE
XCHUNKS = N // CH
XTAIL = N - XCHUNKS * CH
ACC = 3 * B * ED
RSTRIDE = B * ED + 1


def _sc_body(src_hbm, attr_hbm, batch_hbm, x_hbm, z128_hbm,
             xsum_hbm, acc_hbm,
             batch_v, src_v0, src_v1, attr_v0, attr_v1, x_v,
             bidx_v, bidx_t, x_t, acc_v, erep_v,
             sem_s0, sem_s1, sem_a0, sem_a1,
             xsum_s):
    cid = lax.axis_index("c")
    sid = lax.axis_index("s")
    wid = sid * NC + cid
    lane = lax.broadcasted_iota(jnp.int32, (L,), 0)
    ones16 = jnp.full((L,), 1.0, jnp.float32)
    src_b = (src_v0, src_v1)
    attr_b = (attr_v0, attr_v1)
    sem_s = (sem_s0, sem_s1)
    sem_a = (sem_a0, sem_a1)

    pltpu.sync_copy(batch_hbm, batch_v)

    def zstep(i, c):
        acc_v[pl.ds(i * L, L)] = jnp.zeros((L,), jnp.float32)
        return c
    lax.fori_loop(0, ACC // L, zstep, 0)

    def zrstep(i, c):
        erep_v[pl.ds(i * L, L)] = jnp.zeros((L,), jnp.float32)
        return c
    lax.fori_loop(0, L * RSTRIDE // L, zrstep, 0)

    @pl.when(sid == 0)
    def _():
        pltpu.sync_copy(z128_hbm, xsum_s)

    plsc.subcore_barrier()

    ne = ECHUNKS // NW + (wid < ECHUNKS % NW).astype(jnp.int32)

    def estart(k, b):
        base = (wid + k * NW) * CHE
        pltpu.async_copy(src_hbm.at[pl.ds(base, CHE)], src_b[b], sem_s[b])
        pltpu.async_copy(attr_hbm.at[:, pl.ds(base, CHE)], attr_b[b], sem_a[b])

    estart(0, 0)
    estart(1, 1)

    def echunk(i, carry):
        for b in range(2):
            k = i * 2 + b

            @pl.when(k < ne)
            def _():
                pltpu.make_async_copy(src_hbm.at[pl.ds(0, CHE)],
                                      src_b[b], sem_s[b]).wait()
                pltpu.make_async_copy(attr_hbm.at[:, pl.ds(0, CHE)],
                                      attr_b[b], sem_a[b]).wait()
                for g in range(CHE // L):
                    sv = src_b[b][pl.ds(g * L, L)]
                    seg = plsc.load_gather(batch_v, [sv])
                    seg16 = seg * ED
                    plsc.addupdate_scatter(acc_v, [seg16 + lane + B * ED],
                                           ones16)
                    base = lane * RSTRIDE + seg16
                    for f in range(ED):
                        vals = attr_b[b][f, pl.ds(g * L, L)]
                        plsc.addupdate_scatter(erep_v, [base + f], vals)

                @pl.when(k + 2 < ne)
                def _():
                    estart(k + 2, b)
        return carry

    lax.fori_loop(0, (ne + 1) // 2, echunk, 0)

    def rstep(s, c):
        tot = jnp.zeros((L,), jnp.float32)
        for l in range(L):
            tot = tot + plsc.load_gather(erep_v, [l * RSTRIDE + s * L + lane])
        acc_v[pl.ds(s * L, L)] = tot
        return c
    lax.fori_loop(0, B * ED // L, rstep, 0)

    def xchunk(i, carry):
        base = (wid + i * NW) * CH
        pltpu.sync_copy(batch_hbm.at[pl.ds(base, CH)], bidx_v)
        pltpu.sync_copy(x_hbm.at[pl.ds(base, CH)], x_v)
        pltpu.sync_copy(x_v, xsum_s.at[bidx_v], add=True)
        for g in range(CH // L):
            bv = bidx_v[pl.ds(g * L, L)]
            plsc.addupdate_scatter(acc_v, [bv * ED + lane + 2 * B * ED], ones16)
        return carry

    nx = XCHUNKS // NW + (wid < XCHUNKS % NW).astype(jnp.int32)
    lax.fori_loop(0, nx, xchunk, 0)

    @pl.when(wid == 0)
    def _():
        pltpu.sync_copy(batch_hbm.at[pl.ds(XCHUNKS * CH, XTAIL)], bidx_t)
        pltpu.sync_copy(x_hbm.at[pl.ds(XCHUNKS * CH, XTAIL)], x_t)
        pltpu.sync_copy(x_t, xsum_s.at[bidx_t], add=True)
        bv = bidx_t[...]
        plsc.addupdate_scatter(acc_v, [bv * ED + lane + 2 * B * ED], ones16)

    pltpu.sync_copy(acc_v, acc_hbm.at[wid])

    plsc.subcore_barrier()

    @pl.when(sid == 0)
    def _():
        pltpu.sync_copy(xsum_s, xsum_hbm.at[cid])


def _reduce_body(acc_ref, out_ref):
    out_ref[...] = jnp.sum(acc_ref[...], axis=0, keepdims=True)


def _mlp_body(esum_ref, ecnt_ref, xcnt_ref, xsum_ref, u_ref,
              W1_ref, b1_ref, W2_ref, b2_ref, lnw_ref, lnb_ref, out_ref):
    xsum = xsum_ref[0] + xsum_ref[1]
    ecnt = jnp.maximum(jnp.sum(ecnt_ref[...], axis=1, keepdims=True), 1.0)
    xcnt = jnp.maximum(jnp.sum(xcnt_ref[...], axis=1, keepdims=True), 1.0)
    e_mean = esum_ref[...] / ecnt
    x_mean = xsum / xcnt
    u = u_ref[...]
    h = (jnp.dot(u, W1_ref[0:GD, :], preferred_element_type=jnp.float32)
         + jnp.dot(e_mean, W1_ref[GD:GD + ED, :], preferred_element_type=jnp.float32)
         + jnp.dot(x_mean, W1_ref[GD + ED:GD + ED + ND, :], preferred_element_type=jnp.float32)
         + b1_ref[...])
    h = jnp.maximum(h, 0.0)
    o = jnp.dot(h, W2_ref[...], preferred_element_type=jnp.float32) + b2_ref[...] + u
    mu = jnp.mean(o, axis=-1, keepdims=True)
    d = o - mu
    var = jnp.mean(d * d, axis=-1, keepdims=True)
    out_ref[...] = d * jax.lax.rsqrt(var + 1e-5) * lnw_ref[...] + lnb_ref[...]


def _sc_stage(src, edge_attr, batch, x):
    z128 = jnp.zeros((B, ND), jnp.float32)
    mesh = plsc.VectorSubcoreMesh(core_axis_name="c", subcore_axis_name="s",
                                  num_cores=NC, num_subcores=NS)
    f32 = jnp.float32
    sc = functools.partial(
        pl.kernel,
        out_type=[jax.ShapeDtypeStruct((NC, B, ND), f32),
                  jax.ShapeDtypeStruct((NW, ACC), f32)],
        mesh=mesh,
        compiler_params=pltpu.CompilerParams(needs_layout_passes=False),
        scratch_types=[
            pltpu.VMEM((N,), jnp.int32),
            pltpu.VMEM((CHE,), jnp.int32),
            pltpu.VMEM((CHE,), jnp.int32),
            pltpu.VMEM((ED, CHE), f32),
            pltpu.VMEM((ED, CHE), f32),
            pltpu.VMEM((CH, ND), f32),
            pltpu.VMEM((CH,), jnp.int32),
            pltpu.VMEM((XTAIL,), jnp.int32),
            pltpu.VMEM((XTAIL, ND), f32),
            pltpu.VMEM((ACC,), f32),
            pltpu.VMEM((L * RSTRIDE,), f32),
            pltpu.SemaphoreType.DMA,
            pltpu.SemaphoreType.DMA,
            pltpu.SemaphoreType.DMA,
            pltpu.SemaphoreType.DMA,
            pltpu.VMEM_SHARED((B, ND), f32),
        ],
    )(_sc_body)
    return sc(src, edge_attr, batch, x, z128)


def kernel(x, edge_index, edge_attr, u, batch, W1, b1, W2, b2, ln_w, ln_b):
    xsum, acc = _sc_stage(edge_index[0], edge_attr.T, batch, x)

    red = pl.pallas_call(
        _reduce_body,
        out_shape=jax.ShapeDtypeStruct((1, ACC), jnp.float32),
    )(acc)
    esum = red[0, 0:B * ED].reshape(B, ED)
    ecnt = red[0, B * ED:2 * B * ED].reshape(B, ED)
    xcnt = red[0, 2 * B * ED:3 * B * ED].reshape(B, ED)

    out = pl.pallas_call(
        _mlp_body,
        out_shape=jax.ShapeDtypeStruct((B, GD), jnp.float32),
    )(esum, ecnt, xcnt, xsum, u,
      W1, b1.reshape(1, HID), W2, b2.reshape(1, GD),
      ln_w.reshape(1, GD), ln_b.reshape(1, GD))
    return out
